# baseline (speedup 1.0000x reference)
import functools

import jax
import jax.numpy as jnp
from jax import lax
from jax.experimental import pallas as pl
from jax.experimental.pallas import tpu as pltpu
from jax.experimental.pallas import tpu_sc as plsc

B = 4096
L = 200
D = 128
N = B * L
NC = 2
NS = 16
NW = NC * NS
PER_W = N // NW
CHUNK = 128
NCHUNK = PER_W // CHUNK
NQUAD = NCHUNK // 4

_mesh = plsc.VectorSubcoreMesh(core_axis_name="c", subcore_axis_name="s")


@functools.partial(
    pl.kernel,
    mesh=_mesh,
    out_type=jax.ShapeDtypeStruct((N, D), jnp.float32),
    scratch_types=(
        [pltpu.VMEM((PER_W,), jnp.int32)]
        + [pltpu.VMEM((CHUNK, D), jnp.float32) for _ in range(4)]
        + [pltpu.VMEM_SHARED((NS * CHUNK, D), jnp.float32) for _ in range(2)]
        + [pltpu.SemaphoreType.DMA for _ in range(10)]
    ),
)
def _gather_kernel(idx_hbm, table_hbm, out_hbm, idx_v, *rest):
    b = rest[0:4]
    sp = rest[4:6]
    gs = rest[6:10]
    cs = rest[10:14]
    ws = rest[14:16]
    sid = lax.axis_index("s")
    wid = sid * NC + lax.axis_index("c")
    base = wid * PER_W
    s = [sp[i].at[pl.ds(sid * CHUNK, CHUNK)] for i in range(2)]
    pltpu.sync_copy(idx_hbm.at[pl.ds(base, PER_W)], idx_v)
    for q in range(4):
        pltpu.async_copy(
            table_hbm.at[idx_v.at[pl.ds(q * CHUNK, CHUNK)]], b[q], gs[q])

    def body(j, carry):
        g0 = j * 4
        for q in range(4):
            g = g0 + q
            qp = (q - 1) % 4
            pltpu.make_async_copy(
                table_hbm.at[idx_v.at[pl.ds(g * CHUNK, CHUNK)]], b[q],
                gs[q]).wait()

            def wait_w(q=q, g=g):
                pltpu.make_async_copy(
                    s[q % 2],
                    out_hbm.at[pl.ds(base + (g - 2) * CHUNK, CHUNK)],
                    ws[q % 2]).wait()

            if q >= 2:
                wait_w()
            else:
                pl.when(j > 0)(wait_w)

            pltpu.async_copy(b[q], s[q % 2], cs[q])

            def tail(issue_gather, q=q, qp=qp, g=g):
                pltpu.make_async_copy(b[qp], s[(q - 1) % 2], cs[qp]).wait()
                pltpu.async_copy(
                    s[(q - 1) % 2],
                    out_hbm.at[pl.ds(base + (g - 1) * CHUNK, CHUNK)],
                    ws[(q - 1) % 2])

                def issue(qp=qp, g=g):
                    pltpu.async_copy(
                        table_hbm.at[idx_v.at[pl.ds((g + 3) * CHUNK, CHUNK)]],
                        b[qp], gs[qp])

                if issue_gather is None:
                    issue()
                else:
                    pl.when(issue_gather)(issue)

            if q == 0:
                pl.when(j > 0)(functools.partial(tail, None))
            else:
                tail(j + 1 < NQUAD)
        return carry

    lax.fori_loop(0, NQUAD, body, 0)
    gl = NCHUNK - 1
    pltpu.make_async_copy(b[gl % 4], s[gl % 2], cs[gl % 4]).wait()
    pltpu.async_copy(
        s[gl % 2], out_hbm.at[pl.ds(base + gl * CHUNK, CHUNK)], ws[gl % 2])
    for g in (NCHUNK - 2, NCHUNK - 1):
        pltpu.make_async_copy(
            s[g % 2], out_hbm.at[pl.ds(base + g * CHUNK, CHUNK)],
            ws[g % 2]).wait()


def kernel(x, table):
    out = _gather_kernel(x.reshape(-1), table)
    return out.reshape(B, L, D)
